# baseline (speedup 1.0000x reference)
import functools
import math

import jax
import jax.numpy as jnp
from jax import lax
from jax.experimental import pallas as pl
from jax.experimental.pallas import tpu as pltpu
from jax.experimental.pallas import tpu_sc as plsc

EMB_D = 128
NUM_EMB = 22
PAD_IDX = 21
SCALE = math.sqrt(float(EMB_D))

NUM_CORES = 2
NUM_SUBCORES = 16
NUM_WORKERS = NUM_CORES * NUM_SUBCORES

TOTAL = 4096 * 200
PER_WORKER = TOTAL // NUM_WORKERS
CHUNK = 128
NUM_CHUNKS = PER_WORKER // CHUNK
NUM_PAIRS = NUM_CHUNKS // 2
BUF_ROWS = 2 * CHUNK
LANES = 16

_sc_mesh = plsc.VectorSubcoreMesh(core_axis_name="c", subcore_axis_name="s")


@functools.partial(
    pl.kernel,
    mesh=_sc_mesh,
    compiler_params=pltpu.CompilerParams(needs_layout_passes=False),
    out_type=jax.ShapeDtypeStruct((TOTAL * EMB_D,), jnp.float32),
    scratch_types=[
        pltpu.VMEM((NUM_CHUNKS, CHUNK), jnp.int32),
        pltpu.VMEM((BUF_ROWS * EMB_D,), jnp.float32),
        pltpu.VMEM((BUF_ROWS * EMB_D,), jnp.float32),
        pltpu.VMEM((BUF_ROWS * EMB_D,), jnp.float32),
        pltpu.VMEM((NUM_EMB * EMB_D,), jnp.float32),
        pltpu.SemaphoreType.DMA,
    ],
)
def _emb_kernel(table_hbm, idx_hbm, out_hbm, idx_v, rows0, rows1, rows2, tab_v, ssem):
    rings = (rows0, rows1, rows2)
    wid = lax.axis_index("s") * NUM_CORES + lax.axis_index("c")
    base = wid * PER_WORKER

    pltpu.sync_copy(table_hbm, tab_v)
    for e in range(NUM_EMB * EMB_D // LANES):
        sl = pl.ds(e * LANES, LANES)
        if e >= PAD_IDX * EMB_D // LANES:
            tab_v[sl] = jnp.zeros((LANES,), jnp.float32)
        else:
            tab_v[sl] = tab_v[sl] * SCALE

    pltpu.sync_copy(idx_hbm.at[pl.ds(wid * NUM_CHUNKS, NUM_CHUNKS)], idx_v)

    row_iota = lax.iota(jnp.int32, LANES)
    dst_iota = row_iota * EMB_D

    def fill_pair(p, g):
        buf = rings[g]
        for half in range(2):
            ch = 2 * p + half

            def cblock(cb, carry):
                cbase = cb * LANES
                for rg in range(CHUNK // LANES):
                    idx16 = idx_v[ch, pl.ds(rg * LANES, LANES)]
                    src0 = idx16 * EMB_D + cbase
                    dst0 = dst_iota + ((half * CHUNK + rg * LANES) * EMB_D + cbase)
                    for cc in range(LANES):
                        v = plsc.load_gather(tab_v, [src0 + cc])
                        plsc.store_scatter(buf, [dst0 + cc], v)
                return carry

            lax.fori_loop(0, EMB_D // LANES, cblock, 0)

    def scatter_pair(p, g):
        off = (base + p * BUF_ROWS) * EMB_D
        pltpu.async_copy(rings[g], out_hbm.at[pl.ds(off, BUF_ROWS * EMB_D)], ssem)

    def drain_scatters(g):
        pltpu.make_async_copy(
            rings[g], out_hbm.at[pl.ds(0, BUF_ROWS * EMB_D)], ssem
        ).wait()

    def process_pair(p, gi, drain_prev):
        fill_pair(p, gi)
        scatter_pair(p, gi)
        if drain_prev:
            drain_scatters((gi - 1) % 3)

    process_pair(0, 0, False)

    def body(c, carry):
        p = 3 * c + 1
        process_pair(p, 1, True)
        process_pair(p + 1, 2, True)
        process_pair(p + 2, 0, True)
        return carry

    lax.fori_loop(0, (NUM_PAIRS - 4) // 3, body, 0)
    process_pair(NUM_PAIRS - 3, 1, True)
    process_pair(NUM_PAIRS - 2, 2, True)
    process_pair(NUM_PAIRS - 1, 0, True)
    drain_scatters((NUM_PAIRS - 1) % 3)


def kernel(x, table):
    idx = x.reshape(NUM_WORKERS * NUM_CHUNKS, CHUNK).astype(jnp.int32)
    out = _emb_kernel(table.reshape(-1), idx)
    return out.reshape(x.shape[0], x.shape[1], EMB_D)


# speedup vs baseline: 20.4757x; 20.4757x over previous
import functools
import math

import jax
import jax.numpy as jnp
from jax import lax
from jax.experimental import pallas as pl
from jax.experimental.pallas import tpu as pltpu
from jax.experimental.pallas import tpu_sc as plsc

EMB_D = 128
NUM_EMB = 22
PAD_IDX = 21
SCALE = math.sqrt(float(EMB_D))

NUM_CORES = 2
NUM_SUBCORES = 16
NUM_WORKERS = NUM_CORES * NUM_SUBCORES

TOTAL = 4096 * 200
PER_WORKER = TOTAL // NUM_WORKERS
CHUNK = 128
NUM_CHUNKS = PER_WORKER // CHUNK
NUM_PAIRS = NUM_CHUNKS // 2
LANES = 16

_sc_mesh = plsc.VectorSubcoreMesh(core_axis_name="c", subcore_axis_name="s")


@functools.partial(
    pl.kernel,
    mesh=_sc_mesh,
    out_type=jax.ShapeDtypeStruct((TOTAL, EMB_D), jnp.float32),
    scratch_types=[
        pltpu.VMEM((NUM_CHUNKS, CHUNK), jnp.int32),
        pltpu.VMEM((3, 2 * CHUNK, EMB_D), jnp.float32),
        pltpu.VMEM((NUM_EMB, EMB_D), jnp.float32),
        pltpu.VMEM_SHARED((8 * NUM_EMB, EMB_D), jnp.float32),
        pltpu.SemaphoreType.DMA,
        pltpu.SemaphoreType.DMA,
    ],
)
def _emb_kernel(table_hbm, idx_hbm, out_hbm, idx_v, rows_v, tab_v, tab_sh, gsem, ssem):
    wid = lax.axis_index("s") * NUM_CORES + lax.axis_index("c")
    base = wid * PER_WORKER

    @pl.when(lax.axis_index("s") == 0)
    def _stage_table():
        pltpu.sync_copy(table_hbm, tab_v)
        for r in range(NUM_EMB):
            for k in range(EMB_D // LANES):
                sl = pl.ds(k * LANES, LANES)
                if r == PAD_IDX:
                    tab_v[r, sl] = jnp.zeros((LANES,), jnp.float32)
                else:
                    tab_v[r, sl] = tab_v[r, sl] * SCALE
        for rep in range(8):
            pltpu.sync_copy(tab_v, tab_sh.at[pl.ds(rep * NUM_EMB, NUM_EMB)])

    plsc.subcore_barrier()

    def fire_pair(p, g):
        pltpu.async_copy(tab_sh.at[idx_v.at[2 * p]], rows_v.at[g, pl.ds(0, CHUNK)], gsem)
        pltpu.async_copy(
            tab_sh.at[idx_v.at[2 * p + 1]], rows_v.at[g, pl.ds(CHUNK, CHUNK)], gsem
        )

    def drain_gathers(g):
        pltpu.make_async_copy(
            out_hbm.at[pl.ds(0, CHUNK)], rows_v.at[g, pl.ds(0, CHUNK)], gsem
        ).wait()
        pltpu.make_async_copy(
            out_hbm.at[pl.ds(0, CHUNK)], rows_v.at[g, pl.ds(CHUNK, CHUNK)], gsem
        ).wait()

    def scatter_pair(p, g):
        off = base + p * (2 * CHUNK)
        pltpu.async_copy(rows_v.at[g], out_hbm.at[pl.ds(off, 2 * CHUNK)], ssem)

    def drain_scatters(g):
        pltpu.make_async_copy(rows_v.at[g], out_hbm.at[pl.ds(0, 2 * CHUNK)], ssem).wait()

    def process_pair(p, gi, fire_next, drain_prev):
        if fire_next:
            fire_pair(p + 1, (gi + 1) % 3)
        drain_gathers(gi)
        scatter_pair(p, gi)
        if drain_prev:
            drain_scatters((gi - 1) % 3)

    pltpu.sync_copy(idx_hbm.at[pl.ds(wid * NUM_CHUNKS, NUM_CHUNKS)], idx_v)
    rep_off = lax.rem(lax.axis_index("s"), 8) * NUM_EMB

    def bias_body(ch, carry):
        for k in range(CHUNK // LANES):
            sl = pl.ds(k * LANES, LANES)
            idx_v[ch, sl] = idx_v[ch, sl] + rep_off
        return carry

    lax.fori_loop(0, NUM_CHUNKS, bias_body, 0)
    fire_pair(0, 0)
    process_pair(0, 0, True, False)

    def body(c, carry):
        p = 3 * c + 1
        process_pair(p, 1, True, True)
        process_pair(p + 1, 2, True, True)
        process_pair(p + 2, 0, True, True)
        return carry

    lax.fori_loop(0, (NUM_PAIRS - 4) // 3, body, 0)
    process_pair(NUM_PAIRS - 3, 1, True, True)
    process_pair(NUM_PAIRS - 2, 2, True, True)
    process_pair(NUM_PAIRS - 1, 0, False, True)
    drain_scatters((NUM_PAIRS - 1) % 3)


def kernel(x, table):
    idx = x.reshape(NUM_WORKERS * NUM_CHUNKS, CHUNK).astype(jnp.int32)
    out = _emb_kernel(table, idx)
    return out.reshape(x.shape[0], x.shape[1], EMB_D)


# speedup vs baseline: 20.5557x; 1.0039x over previous
import functools
import math

import jax
import jax.numpy as jnp
from jax import lax
from jax.experimental import pallas as pl
from jax.experimental.pallas import tpu as pltpu
from jax.experimental.pallas import tpu_sc as plsc

EMB_D = 128
NUM_EMB = 22
PAD_IDX = 21
SCALE = math.sqrt(float(EMB_D))

NUM_CORES = 2
NUM_SUBCORES = 16
NUM_WORKERS = NUM_CORES * NUM_SUBCORES

TOTAL = 4096 * 200
PER_WORKER = TOTAL // NUM_WORKERS
CHUNK = 128
NUM_CHUNKS = PER_WORKER // CHUNK
NUM_PAIRS = NUM_CHUNKS // 2
LANES = 16

_sc_mesh = plsc.VectorSubcoreMesh(core_axis_name="c", subcore_axis_name="s")


@functools.partial(
    pl.kernel,
    mesh=_sc_mesh,
    out_type=jax.ShapeDtypeStruct((TOTAL, EMB_D), jnp.float32),
    scratch_types=[
        pltpu.VMEM((NUM_CHUNKS, CHUNK), jnp.int32),
        pltpu.VMEM((3, 2 * CHUNK, EMB_D), jnp.float32),
        pltpu.VMEM((NUM_EMB, EMB_D), jnp.float32),
        pltpu.VMEM_SHARED((NUM_EMB, EMB_D), jnp.float32),
        pltpu.SemaphoreType.DMA,
        pltpu.SemaphoreType.DMA,
    ],
)
def _emb_kernel(table_hbm, idx_hbm, out_hbm, idx_v, rows_v, tab_v, tab_sh, gsem, ssem):
    wid = lax.axis_index("s") * NUM_CORES + lax.axis_index("c")
    base = wid * PER_WORKER

    @pl.when(lax.axis_index("s") == 0)
    def _stage_table():
        pltpu.sync_copy(table_hbm, tab_v)
        for r in range(NUM_EMB):
            for k in range(EMB_D // LANES):
                sl = pl.ds(k * LANES, LANES)
                if r == PAD_IDX:
                    tab_v[r, sl] = jnp.zeros((LANES,), jnp.float32)
                else:
                    tab_v[r, sl] = tab_v[r, sl] * SCALE
        pltpu.sync_copy(tab_v, tab_sh)

    plsc.subcore_barrier()

    def fire_pair(p, g):
        pltpu.async_copy(tab_sh.at[idx_v.at[2 * p]], rows_v.at[g, pl.ds(0, CHUNK)], gsem)
        pltpu.async_copy(
            tab_sh.at[idx_v.at[2 * p + 1]], rows_v.at[g, pl.ds(CHUNK, CHUNK)], gsem
        )

    def drain_gathers(g):
        pltpu.make_async_copy(
            out_hbm.at[pl.ds(0, CHUNK)], rows_v.at[g, pl.ds(0, CHUNK)], gsem
        ).wait()
        pltpu.make_async_copy(
            out_hbm.at[pl.ds(0, CHUNK)], rows_v.at[g, pl.ds(CHUNK, CHUNK)], gsem
        ).wait()

    def scatter_pair(p, g):
        off = base + p * (2 * CHUNK)
        pltpu.async_copy(rows_v.at[g], out_hbm.at[pl.ds(off, 2 * CHUNK)], ssem)

    def drain_scatters(g):
        pltpu.make_async_copy(rows_v.at[g], out_hbm.at[pl.ds(0, 2 * CHUNK)], ssem).wait()

    def process_pair(p, gi, fire_next, drain_prev):
        if fire_next:
            fire_pair(p + 1, (gi + 1) % 3)
        drain_gathers(gi)
        scatter_pair(p, gi)
        if drain_prev:
            drain_scatters((gi - 1) % 3)

    pltpu.sync_copy(idx_hbm.at[pl.ds(wid * NUM_CHUNKS, NUM_CHUNKS)], idx_v)
    fire_pair(0, 0)
    process_pair(0, 0, True, False)

    def body(c, carry):
        p = 3 * c + 1
        process_pair(p, 1, True, True)
        process_pair(p + 1, 2, True, True)
        process_pair(p + 2, 0, True, True)
        return carry

    lax.fori_loop(0, (NUM_PAIRS - 4) // 3, body, 0)
    process_pair(NUM_PAIRS - 3, 1, True, True)
    process_pair(NUM_PAIRS - 2, 2, True, True)
    process_pair(NUM_PAIRS - 1, 0, False, True)
    drain_scatters((NUM_PAIRS - 1) % 3)


def kernel(x, table):
    idx = x.reshape(NUM_WORKERS * NUM_CHUNKS, CHUNK).astype(jnp.int32)
    out = _emb_kernel(table, idx)
    return out.reshape(x.shape[0], x.shape[1], EMB_D)
